# baseline (speedup 1.0000x reference)
import functools

import jax
import jax.numpy as jnp
from jax import lax
from jax.experimental import pallas as pl
from jax.experimental.pallas import tpu as pltpu
from jax.experimental.pallas import tpu_sc as plsc

N = 10000
NF = 5000
NFP = 5120
F = 128
H = 128
HEADS = 8
K = 2048
PAD_CLUSTER = 2047
ZROW = 2040

NSUB = 16
NCORE = 2
NW = NSUB * NCORE
CHUNK = NFP // NSUB
BLKS = CHUNK // 16
WCHUNK = NFP // NW
WBLKS = WCHUNK // 16
IDX_MINOR = 64
NIDX = CHUNK // IDX_MINOR
GIDX_MINOR = 32
NGIDX = WCHUNK // GIDX_MINOR


def _tca_body(fx_ref, Wa_ref, ba_ref, fh_ref):
    fh = jnp.dot(fx_ref[...], Wa_ref[...], preferred_element_type=jnp.float32)
    fh = jnp.maximum(fh + ba_ref[...], 0.0)
    fh_ref[pl.ds(0, NF), :] = fh
    fh_ref[pl.ds(NF, NFP - NF), :] = jnp.zeros((NFP - NF, H), jnp.float32)


def _tca(frag_x, W_a, b_a):
    return pl.pallas_call(
        _tca_body,
        out_shape=jax.ShapeDtypeStruct((NFP, H), jnp.float32),
    )(frag_x, W_a, b_a.reshape(1, H))


def _tcb_body(x_ref, Wa_ref, ba_ref, g1_ref, b1_ref, Wout_ref, bout_ref,
              mw_ref):
    xa = jnp.dot(x_ref[...], Wa_ref[...], preferred_element_type=jnp.float32)
    xa = xa + ba_ref[...]
    mu = jnp.mean(xa, axis=0, keepdims=True)
    var = jnp.mean((xa - mu) ** 2, axis=0, keepdims=True)
    mol = (xa - mu) / jnp.sqrt(var + 1e-5) * g1_ref[...] + b1_ref[...]
    mol = jnp.maximum(mol, 0.0)
    Wo1 = Wout_ref[pl.ds(0, H), :]
    mw_ref[...] = (jnp.dot(mol, Wo1, preferred_element_type=jnp.float32)
                   + bout_ref[...])


def _tcb(x, W_a, b_a, gamma1, beta1, W_out, b_out):
    return pl.pallas_call(
        _tcb_body,
        out_shape=jax.ShapeDtypeStruct((N, 1), jnp.float32),
    )(x, W_a, b_a.reshape(1, H), gamma1.reshape(1, H), beta1.reshape(1, H),
      W_out, b_out.reshape(1, 1))


def _sc_mid_body(fh_hbm, ci_hbm, G_hbm,
                 ci_v, fh_v, rows_v, sidx_v, gidx_v, zb_v, fp_sh,
                 sem_ci, sem_fh, sem_s, sem_g):
    sid = lax.axis_index("s")
    cid = lax.axis_index("c")
    base = sid * CHUNK
    wid = sid * NCORE + cid
    wbase = wid * WCHUNK
    iota = lax.broadcasted_iota(jnp.int32, (16,), 0)

    a_ci = pltpu.async_copy(ci_hbm, ci_v.at[pl.ds(0, NF)], sem_ci)
    a_fh = pltpu.async_copy(fh_hbm.at[pl.ds(base, CHUNK)], fh_v, sem_fh)
    for rr in range(16):
        for kk in range(H // 16):
            zb_v[rr, pl.ds(kk * 16, 16)] = jnp.zeros((16,), jnp.float32)
    for rr in range(K // NSUB // 16):
        pltpu.sync_copy(zb_v, fp_sh.at[pl.ds(sid * (K // NSUB) + rr * 16, 16)])

    a_ci.wait()
    sent = jnp.full((16,), PAD_CLUSTER, jnp.int32)
    for k in range((NFP - NF) // 16):
        plsc.store_scatter(ci_v, [NF + 16 * k + iota], sent)
    plsc.store_scatter(ci_v, [NFP - 16 + iota], sent)
    plsc.store_scatter(ci_v, [NFP + iota], sent)

    for b in range(BLKS):
        v = ci_v[pl.ds(base + 16 * b, 16)]
        sidx_v[b // (IDX_MINOR // 16), pl.ds((b % (IDX_MINOR // 16)) * 16, 16)] = v

    a_fh.wait()
    plsc.subcore_barrier()

    scat = [pltpu.async_copy(fh_v.at[pl.ds(j * IDX_MINOR, IDX_MINOR)],
                             fp_sh.at[sidx_v.at[j]], sem_s, add=True)
            for j in range(NIDX)]

    def _pref(b, acc):
        v = ci_v[pl.ds(16 * b, 16)]
        nxt = plsc.load_gather(ci_v, [16 * b + 1 + iota])
        return acc + jnp.where(v != nxt, 1, 0).astype(jnp.int32)

    acc0 = jnp.zeros((16,), jnp.int32)
    acc = lax.fori_loop(0, wid * WBLKS, _pref, acc0)
    carry = jnp.sum(acc)

    for b in range(WBLKS):
        j = wbase + 16 * b + iota
        v = ci_v[pl.ds(wbase + 16 * b, 16)]
        nxt = plsc.load_gather(ci_v, [wbase + 16 * b + 1 + iota])
        last = v != nxt
        lasti = jnp.where(last, 1, 0).astype(jnp.int32)
        cs = plsc.cumsum(lasti)
        inv = carry + cs - lasti
        mask = last & (j < NF)
        gidx = jnp.where(mask, inv, ZROW)
        gidx_v[b // (GIDX_MINOR // 16), pl.ds((b % (GIDX_MINOR // 16)) * 16, 16)] = gidx
        carry = carry + jnp.sum(lasti)

    for a in scat:
        a.wait()
    plsc.subcore_barrier()

    gath = [pltpu.async_copy(fp_sh.at[gidx_v.at[j]],
                             rows_v.at[pl.ds(j * GIDX_MINOR, GIDX_MINOR)],
                             sem_g)
            for j in range(NGIDX)]
    for a in gath:
        a.wait()
    pltpu.sync_copy(rows_v, G_hbm.at[pl.ds(wbase, WCHUNK)])


def _sc_mid(fh_pad, ci_raw):
    mesh = plsc.VectorSubcoreMesh(core_axis_name="c", subcore_axis_name="s")
    fn = functools.partial(
        pl.kernel,
        mesh=mesh,
        compiler_params=pltpu.CompilerParams(needs_layout_passes=False),
        out_type=jax.ShapeDtypeStruct((NFP, H), jnp.float32),
        scratch_types=[
            pltpu.VMEM((NFP + 16,), jnp.int32),
            pltpu.VMEM((CHUNK, H), jnp.float32),
            pltpu.VMEM((WCHUNK, H), jnp.float32),
            pltpu.VMEM((NIDX, IDX_MINOR), jnp.int32),
            pltpu.VMEM((NGIDX, GIDX_MINOR), jnp.int32),
            pltpu.VMEM((16, H), jnp.float32),
            pltpu.VMEM_SHARED((K, H), jnp.float32),
            pltpu.SemaphoreType.DMA,
            pltpu.SemaphoreType.DMA,
            pltpu.SemaphoreType.DMA,
            pltpu.SemaphoreType.DMA,
        ],
    )(_sc_mid_body)
    return fn(fh_pad, ci_raw)


def _tc2_body(mw_ref, G_ref, Wsrc_ref, bgat_ref, Wout_ref, out_ref):
    G = jnp.maximum(G_ref[...], 0.0)
    r = jax.lax.broadcasted_iota(jnp.int32, (HEADS * H, H), 0)
    c = jax.lax.broadcasted_iota(jnp.int32, (HEADS * H, H), 1)
    S = jnp.where(r % H == c, 1.0 / HEADS, 0.0).astype(jnp.float32)
    W_mean = jnp.dot(Wsrc_ref[...], S, preferred_element_type=jnp.float32)
    s_mean = jnp.dot(G, W_mean, preferred_element_type=jnp.float32)
    frag_m = jnp.maximum(s_mean + bgat_ref[...], 0.0)
    cb = jnp.maximum(bgat_ref[...], 0.0)
    Wo2 = Wout_ref[pl.ds(H, H), :]
    top = jnp.dot(frag_m, Wo2, preferred_element_type=jnp.float32)
    cb2 = jnp.dot(cb, Wo2, preferred_element_type=jnp.float32)
    rows = jax.lax.broadcasted_iota(jnp.int32, (N, 1), 0)
    top_full = jnp.concatenate(
        [top, jnp.zeros((N - NFP, 1), jnp.float32)], axis=0)
    out_ref[...] = mw_ref[...] + jnp.where(rows < NFP, top_full, cb2)


def _tc2(mw, G, W_gat_src, b_gat, W_out):
    return pl.pallas_call(
        _tc2_body,
        out_shape=jax.ShapeDtypeStruct((N, 1), jnp.float32),
    )(mw, G, W_gat_src, b_gat.reshape(1, H), W_out)


def kernel(x, edge_index, edge_attr, frag_x, frag_edge_index, frag_edge_attr,
           cluster_index, W_a, b_a, gamma1, beta1, W_b, b_b, gamma2, beta2,
           W_gat_src, W_gat_dst, att_src, att_dst, b_gat, W_out, b_out):
    fh_pad = _tca(frag_x, W_a, b_a)
    G = _sc_mid(fh_pad, cluster_index.astype(jnp.int32))
    mw = _tcb(x, W_a, b_a, gamma1, beta1, W_out, b_out)
    return _tc2(mw, G, W_gat_src, b_gat, W_out)


# speedup vs baseline: 1.0563x; 1.0563x over previous
import functools

import jax
import jax.numpy as jnp
from jax import lax
from jax.experimental import pallas as pl
from jax.experimental.pallas import tpu as pltpu
from jax.experimental.pallas import tpu_sc as plsc

N = 10000
NF = 5000
NFP = 5120
F = 128
H = 128
HEADS = 8
K = 2048
PAD_CLUSTER = 2047
ZROW = 2040

NSUB = 16
CHUNK = NFP // NSUB
BLKS = CHUNK // 16
IDX_MINOR = 64
NIDX = CHUNK // IDX_MINOR


def _tca_body(fx_ref, Wa_ref, ba_ref, fh_ref):
    fh = jnp.dot(fx_ref[...], Wa_ref[...], preferred_element_type=jnp.float32)
    fh = jnp.maximum(fh + ba_ref[...], 0.0)
    fh_ref[pl.ds(0, NF), :] = fh
    fh_ref[pl.ds(NF, NFP - NF), :] = jnp.zeros((NFP - NF, H), jnp.float32)


def _tca(frag_x, W_a, b_a):
    return pl.pallas_call(
        _tca_body,
        out_shape=jax.ShapeDtypeStruct((NFP, H), jnp.float32),
    )(frag_x, W_a, b_a.reshape(1, H))


def _tcb_body(x_ref, Wa_ref, ba_ref, g1_ref, b1_ref, Wout_ref, bout_ref,
              mw_ref):
    xa = jnp.dot(x_ref[...], Wa_ref[...], preferred_element_type=jnp.float32)
    xa = xa + ba_ref[...]
    mu = jnp.mean(xa, axis=0, keepdims=True)
    var = jnp.mean((xa - mu) ** 2, axis=0, keepdims=True)
    mol = (xa - mu) / jnp.sqrt(var + 1e-5) * g1_ref[...] + b1_ref[...]
    mol = jnp.maximum(mol, 0.0)
    Wo1 = Wout_ref[pl.ds(0, H), :]
    mw_ref[...] = (jnp.dot(mol, Wo1, preferred_element_type=jnp.float32)
                   + bout_ref[...])


def _tcb(x, W_a, b_a, gamma1, beta1, W_out, b_out):
    return pl.pallas_call(
        _tcb_body,
        out_shape=jax.ShapeDtypeStruct((N, 1), jnp.float32),
    )(x, W_a, b_a.reshape(1, H), gamma1.reshape(1, H), beta1.reshape(1, H),
      W_out, b_out.reshape(1, 1))


def _sc_mid_body(fh_hbm, ci_hbm, G_hbm,
                 ci_v, fh_v, rows_v, sidx_v, gidx_v, zb_v, fp_sh,
                 sem_ci, sem_fh, sem_s, sem_g):
    sid = lax.axis_index("s")
    base = sid * CHUNK
    iota = lax.broadcasted_iota(jnp.int32, (16,), 0)

    a_ci = pltpu.async_copy(ci_hbm, ci_v.at[pl.ds(0, NF)], sem_ci)
    a_fh = pltpu.async_copy(fh_hbm.at[pl.ds(base, CHUNK)], fh_v, sem_fh)
    for rr in range(16):
        for kk in range(H // 16):
            zb_v[rr, pl.ds(kk * 16, 16)] = jnp.zeros((16,), jnp.float32)
    for rr in range(K // NSUB // 16):
        pltpu.sync_copy(zb_v, fp_sh.at[pl.ds(sid * (K // NSUB) + rr * 16, 16)])

    a_ci.wait()
    sent = jnp.full((16,), PAD_CLUSTER, jnp.int32)
    for k in range((NFP - NF) // 16):
        plsc.store_scatter(ci_v, [NF + 16 * k + iota], sent)
    plsc.store_scatter(ci_v, [NFP - 16 + iota], sent)
    plsc.store_scatter(ci_v, [NFP + iota], sent)

    for b in range(BLKS):
        v = ci_v[pl.ds(base + 16 * b, 16)]
        sidx_v[b // (IDX_MINOR // 16), pl.ds((b % (IDX_MINOR // 16)) * 16, 16)] = v

    a_fh.wait()
    plsc.subcore_barrier()

    scat = [pltpu.async_copy(fh_v.at[pl.ds(j * IDX_MINOR, IDX_MINOR)],
                             fp_sh.at[sidx_v.at[j]], sem_s, add=True)
            for j in range(NIDX)]

    def _pref(b4, acc):
        for t in range(4):
            b = 4 * b4 + t
            v = ci_v[pl.ds(16 * b, 16)]
            nxt = plsc.load_gather(ci_v, [16 * b + 1 + iota])
            acc = acc + jnp.where(v != nxt, 1, 0).astype(jnp.int32)
        return acc

    acc0 = jnp.zeros((16,), jnp.int32)
    acc = lax.fori_loop(0, sid * (BLKS // 4), _pref, acc0)
    carry = jnp.sum(acc)

    for b in range(BLKS):
        j = base + 16 * b + iota
        v = ci_v[pl.ds(base + 16 * b, 16)]
        nxt = plsc.load_gather(ci_v, [base + 16 * b + 1 + iota])
        last = v != nxt
        lasti = jnp.where(last, 1, 0).astype(jnp.int32)
        cs = plsc.cumsum(lasti)
        inv = carry + cs - lasti
        mask = last & (j < NF)
        gidx = jnp.where(mask, inv, ZROW)
        gidx_v[b // (IDX_MINOR // 16), pl.ds((b % (IDX_MINOR // 16)) * 16, 16)] = gidx
        carry = carry + jnp.sum(lasti)

    for a in scat:
        a.wait()
    plsc.subcore_barrier()

    gath = [pltpu.async_copy(fp_sh.at[gidx_v.at[j]],
                             rows_v.at[pl.ds(j * IDX_MINOR, IDX_MINOR)], sem_g)
            for j in range(NIDX)]
    for a in gath:
        a.wait()
    pltpu.sync_copy(rows_v, G_hbm.at[pl.ds(base, CHUNK)])


def _sc_mid(fh_pad, ci_raw):
    mesh = plsc.VectorSubcoreMesh(core_axis_name="c", subcore_axis_name="s",
                                  num_cores=1)
    fn = functools.partial(
        pl.kernel,
        mesh=mesh,
        compiler_params=pltpu.CompilerParams(needs_layout_passes=False),
        out_type=jax.ShapeDtypeStruct((NFP, H), jnp.float32),
        scratch_types=[
            pltpu.VMEM((NFP + 16,), jnp.int32),
            pltpu.VMEM((CHUNK, H), jnp.float32),
            pltpu.VMEM((CHUNK, H), jnp.float32),
            pltpu.VMEM((NIDX, IDX_MINOR), jnp.int32),
            pltpu.VMEM((NIDX, IDX_MINOR), jnp.int32),
            pltpu.VMEM((16, H), jnp.float32),
            pltpu.VMEM_SHARED((K, H), jnp.float32),
            pltpu.SemaphoreType.DMA,
            pltpu.SemaphoreType.DMA,
            pltpu.SemaphoreType.DMA,
            pltpu.SemaphoreType.DMA,
        ],
    )(_sc_mid_body)
    return fn(fh_pad, ci_raw)


def _tc2_body(mw_ref, G_ref, Wsrc_ref, bgat_ref, Wout_ref, out_ref):
    G = jnp.maximum(G_ref[...], 0.0)
    r = jax.lax.broadcasted_iota(jnp.int32, (HEADS * H, H), 0)
    c = jax.lax.broadcasted_iota(jnp.int32, (HEADS * H, H), 1)
    S = jnp.where(r % H == c, 1.0 / HEADS, 0.0).astype(jnp.float32)
    W_mean = jnp.dot(Wsrc_ref[...], S, preferred_element_type=jnp.float32)
    s_mean = jnp.dot(G, W_mean, preferred_element_type=jnp.float32)
    frag_m = jnp.maximum(s_mean + bgat_ref[...], 0.0)
    cb = jnp.maximum(bgat_ref[...], 0.0)
    Wo2 = Wout_ref[pl.ds(H, H), :]
    top = jnp.dot(frag_m, Wo2, preferred_element_type=jnp.float32)
    cb2 = jnp.dot(cb, Wo2, preferred_element_type=jnp.float32)
    rows = jax.lax.broadcasted_iota(jnp.int32, (N, 1), 0)
    top_full = jnp.concatenate(
        [top, jnp.zeros((N - NFP, 1), jnp.float32)], axis=0)
    out_ref[...] = mw_ref[...] + jnp.where(rows < NFP, top_full, cb2)


def _tc2(mw, G, W_gat_src, b_gat, W_out):
    return pl.pallas_call(
        _tc2_body,
        out_shape=jax.ShapeDtypeStruct((N, 1), jnp.float32),
    )(mw, G, W_gat_src, b_gat.reshape(1, H), W_out)


def kernel(x, edge_index, edge_attr, frag_x, frag_edge_index, frag_edge_attr,
           cluster_index, W_a, b_a, gamma1, beta1, W_b, b_b, gamma2, beta2,
           W_gat_src, W_gat_dst, att_src, att_dst, b_gat, W_out, b_out):
    fh_pad = _tca(frag_x, W_a, b_a)
    G = _sc_mid(fh_pad, cluster_index.astype(jnp.int32))
    mw = _tcb(x, W_a, b_a, gamma1, beta1, W_out, b_out)
    return _tc2(mw, G, W_gat_src, b_gat, W_out)


# speedup vs baseline: 1.0652x; 1.0084x over previous
import functools

import jax
import jax.numpy as jnp
from jax import lax
from jax.experimental import pallas as pl
from jax.experimental.pallas import tpu as pltpu
from jax.experimental.pallas import tpu_sc as plsc

N = 10000
NF = 5000
NFP = 5120
F = 128
H = 128
HEADS = 8
K = 2048
PAD_CLUSTER = 2047
ZROW = 2040

NSUB = 16
CHUNK = NFP // NSUB
BLKS = CHUNK // 16
IDX_MINOR = 64
NIDX = CHUNK // IDX_MINOR


def _tca_body(fx_ref, Wa_ref, ba_ref, fh_ref):
    fh = jnp.dot(fx_ref[...], Wa_ref[...], preferred_element_type=jnp.float32)
    fh = jnp.maximum(fh + ba_ref[...], 0.0)
    fh_ref[pl.ds(0, NF), :] = fh
    fh_ref[pl.ds(NF, NFP - NF), :] = jnp.zeros((NFP - NF, H), jnp.float32)


def _tca(frag_x, W_a, b_a):
    return pl.pallas_call(
        _tca_body,
        out_shape=jax.ShapeDtypeStruct((NFP, H), jnp.float32),
    )(frag_x, W_a, b_a.reshape(1, H))


def _tcb_body(x_ref, Wa_ref, ba_ref, g1_ref, b1_ref, Wout_ref, bout_ref,
              mw_ref):
    xa = jnp.dot(x_ref[...], Wa_ref[...], preferred_element_type=jnp.float32)
    xa = xa + ba_ref[...]
    mu = jnp.mean(xa, axis=0, keepdims=True)
    var = jnp.mean((xa - mu) ** 2, axis=0, keepdims=True)
    mol = (xa - mu) / jnp.sqrt(var + 1e-5) * g1_ref[...] + b1_ref[...]
    mol = jnp.maximum(mol, 0.0)
    Wo1 = Wout_ref[pl.ds(0, H), :]
    mw_ref[...] = (jnp.dot(mol, Wo1, preferred_element_type=jnp.float32)
                   + bout_ref[...])


def _tcb(x, W_a, b_a, gamma1, beta1, W_out, b_out):
    return pl.pallas_call(
        _tcb_body,
        out_shape=jax.ShapeDtypeStruct((N, 1), jnp.float32),
    )(x, W_a, b_a.reshape(1, H), gamma1.reshape(1, H), beta1.reshape(1, H),
      W_out, b_out.reshape(1, 1))


def _sc_mid_body(fh_hbm, ci_hbm, G_hbm,
                 ci_v, fh_v, rows_v, sidx_v, gidx_v, zb_v, fp_sh,
                 sem_ci, sem_fh, sem_s, sem_g):
    sid = lax.axis_index("s")
    base = sid * CHUNK
    iota = lax.broadcasted_iota(jnp.int32, (16,), 0)

    a_ci = pltpu.async_copy(ci_hbm, ci_v.at[pl.ds(0, NF)], sem_ci)
    a_fh = pltpu.async_copy(fh_hbm.at[pl.ds(base, CHUNK)], fh_v, sem_fh)
    for rr in range(16):
        for kk in range(H // 16):
            zb_v[rr, pl.ds(kk * 16, 16)] = jnp.zeros((16,), jnp.float32)
    for rr in range(K // NSUB // 16):
        pltpu.sync_copy(zb_v, fp_sh.at[pl.ds(sid * (K // NSUB) + rr * 16, 16)])

    a_ci.wait()
    sent = jnp.full((16,), PAD_CLUSTER, jnp.int32)
    for k in range((NFP - NF) // 16):
        plsc.store_scatter(ci_v, [NF + 16 * k + iota], sent)
    plsc.store_scatter(ci_v, [NFP - 16 + iota], sent)
    plsc.store_scatter(ci_v, [NFP + iota], sent)

    for b in range(BLKS):
        v = ci_v[pl.ds(base + 16 * b, 16)]
        sidx_v[b // (IDX_MINOR // 16), pl.ds((b % (IDX_MINOR // 16)) * 16, 16)] = v

    a_fh.wait()
    plsc.subcore_barrier()

    scat = [pltpu.async_copy(fh_v.at[pl.ds(j * IDX_MINOR, IDX_MINOR)],
                             fp_sh.at[sidx_v.at[j]], sem_s, add=True)
            for j in range(NIDX)]

    def _pref(b4, acc):
        for t in range(4):
            b = 4 * b4 + t
            v = ci_v[pl.ds(16 * b, 16)]
            nxt = plsc.load_gather(ci_v, [16 * b + 1 + iota])
            acc = acc + jnp.where(v != nxt, 1, 0).astype(jnp.int32)
        return acc

    acc0 = jnp.zeros((16,), jnp.int32)
    acc = lax.fori_loop(0, sid * (BLKS // 4), _pref, acc0)
    carry = jnp.sum(acc)

    for b in range(BLKS):
        j = base + 16 * b + iota
        v = ci_v[pl.ds(base + 16 * b, 16)]
        nxt = plsc.load_gather(ci_v, [base + 16 * b + 1 + iota])
        last = v != nxt
        lasti = jnp.where(last, 1, 0).astype(jnp.int32)
        cs = plsc.cumsum(lasti)
        inv = carry + cs - lasti
        mask = last & (j < NF)
        gidx = jnp.where(mask, inv, ZROW)
        gidx_v[b // (IDX_MINOR // 16), pl.ds((b % (IDX_MINOR // 16)) * 16, 16)] = gidx
        carry = carry + jnp.sum(lasti)

    for a in scat:
        a.wait()
    plsc.subcore_barrier()

    gath = [pltpu.async_copy(fp_sh.at[gidx_v.at[j]],
                             rows_v.at[pl.ds(j * IDX_MINOR, IDX_MINOR)], sem_g)
            for j in range(NIDX)]
    wr = []
    for j in range(NIDX):
        gath[j].wait()
        wr.append(pltpu.async_copy(
            rows_v.at[pl.ds(j * IDX_MINOR, IDX_MINOR)],
            G_hbm.at[pl.ds(base + j * IDX_MINOR, IDX_MINOR)], sem_ci))
    for a in wr:
        a.wait()


def _sc_mid(fh_pad, ci_raw):
    mesh = plsc.VectorSubcoreMesh(core_axis_name="c", subcore_axis_name="s",
                                  num_cores=1)
    fn = functools.partial(
        pl.kernel,
        mesh=mesh,
        compiler_params=pltpu.CompilerParams(needs_layout_passes=False),
        out_type=jax.ShapeDtypeStruct((NFP, H), jnp.float32),
        scratch_types=[
            pltpu.VMEM((NFP + 16,), jnp.int32),
            pltpu.VMEM((CHUNK, H), jnp.float32),
            pltpu.VMEM((CHUNK, H), jnp.float32),
            pltpu.VMEM((NIDX, IDX_MINOR), jnp.int32),
            pltpu.VMEM((NIDX, IDX_MINOR), jnp.int32),
            pltpu.VMEM((16, H), jnp.float32),
            pltpu.VMEM_SHARED((K, H), jnp.float32),
            pltpu.SemaphoreType.DMA,
            pltpu.SemaphoreType.DMA,
            pltpu.SemaphoreType.DMA,
            pltpu.SemaphoreType.DMA,
        ],
    )(_sc_mid_body)
    return fn(fh_pad, ci_raw)


def _tc2_body(mw_ref, G_ref, Wsrc_ref, bgat_ref, Wout_ref, out_ref):
    G = jnp.maximum(G_ref[...], 0.0)
    r = jax.lax.broadcasted_iota(jnp.int32, (HEADS * H, H), 0)
    c = jax.lax.broadcasted_iota(jnp.int32, (HEADS * H, H), 1)
    S = jnp.where(r % H == c, 1.0 / HEADS, 0.0).astype(jnp.float32)
    W_mean = jnp.dot(Wsrc_ref[...], S, preferred_element_type=jnp.float32)
    s_mean = jnp.dot(G, W_mean, preferred_element_type=jnp.float32)
    frag_m = jnp.maximum(s_mean + bgat_ref[...], 0.0)
    cb = jnp.maximum(bgat_ref[...], 0.0)
    Wo2 = Wout_ref[pl.ds(H, H), :]
    top = jnp.dot(frag_m, Wo2, preferred_element_type=jnp.float32)
    cb2 = jnp.dot(cb, Wo2, preferred_element_type=jnp.float32)
    rows = jax.lax.broadcasted_iota(jnp.int32, (N, 1), 0)
    top_full = jnp.concatenate(
        [top, jnp.zeros((N - NFP, 1), jnp.float32)], axis=0)
    out_ref[...] = mw_ref[...] + jnp.where(rows < NFP, top_full, cb2)


def _tc2(mw, G, W_gat_src, b_gat, W_out):
    return pl.pallas_call(
        _tc2_body,
        out_shape=jax.ShapeDtypeStruct((N, 1), jnp.float32),
    )(mw, G, W_gat_src, b_gat.reshape(1, H), W_out)


def kernel(x, edge_index, edge_attr, frag_x, frag_edge_index, frag_edge_attr,
           cluster_index, W_a, b_a, gamma1, beta1, W_b, b_b, gamma2, beta2,
           W_gat_src, W_gat_dst, att_src, att_dst, b_gat, W_out, b_out):
    fh_pad = _tca(frag_x, W_a, b_a)
    G = _sc_mid(fh_pad, cluster_index.astype(jnp.int32))
    mw = _tcb(x, W_a, b_a, gamma1, beta1, W_out, b_out)
    return _tc2(mw, G, W_gat_src, b_gat, W_out)
